# Initial kernel scaffold; baseline (speedup 1.0000x reference)
import jax
import jax.numpy as jnp
from jax.experimental import pallas as pl


def kernel(x, edge_index, W1e, g1e, b1e, W2e, bb2e, W1a, g1a, b1a, W2a, bb2a):
    raise NotImplementedError("write your pallas kernel here")


# speedup vs baseline: 4.3991x; 4.3991x over previous
import functools

import jax
import jax.numpy as jnp
from jax import lax
from jax.experimental import pallas as pl
from jax.experimental.pallas import tpu as pltpu
from jax.experimental.pallas import tpu_sc as plsc

EPS = 1e-5

N = 10000
E = 320000
D = 128
NC = 2
NS = 16
NW = NC * NS
EPW = E // NW
CH = 80
NCH = EPW // CH
NV = D // 16
SB = 624
SREM = N - NS * SB

_mesh = plsc.VectorSubcoreMesh(
    core_axis_name="c", subcore_axis_name="s", num_cores=NC, num_subcores=NS)


def _tc_pre_body(x_ref, w_ref, y_ref):
    y_ref[...] = jnp.dot(x_ref[...], w_ref[...],
                         preferred_element_type=jnp.float32)


def _tc_pre(x, w):
    return pl.pallas_call(
        _tc_pre_body,
        out_shape=jax.ShapeDtypeStruct((N, D), jnp.float32),
    )(x, w)


@functools.partial(
    pl.kernel,
    out_type=(jax.ShapeDtypeStruct((E, D), jnp.float32),
              jax.ShapeDtypeStruct((NW, 2 * D), jnp.float32)),
    mesh=_mesh,
    scratch_types=[
        pltpu.VMEM((NCH, CH), jnp.int32),
        pltpu.VMEM((NCH, CH), jnp.int32),
        pltpu.VMEM((CH, D), jnp.float32),
        pltpu.VMEM((CH, D), jnp.float32),
        pltpu.VMEM((CH, D), jnp.float32),
        pltpu.VMEM((2 * D,), jnp.float32),
        pltpu.SemaphoreType.DMA,
        pltpu.SemaphoreType.DMA,
    ],
)
def _sc_pass1(y_hbm, row_hbm, col_hbm, h_hbm, stats_hbm,
              rowi_v, coli_v, yr_v, yc_v, hb_v, st_v, sem1, sem2):
    c = lax.axis_index("c")
    s_ = lax.axis_index("s")
    wid = s_ * NC + c
    ebase = wid * EPW

    pltpu.sync_copy(row_hbm.at[wid], rowi_v)
    pltpu.sync_copy(col_hbm.at[wid], coli_v)

    zero = jnp.zeros((16,), jnp.float32)
    init = tuple(zero for _ in range(2 * NV))

    def chunk_body(j, acc):
        cp1 = pltpu.async_copy(y_hbm.at[rowi_v.at[j]], yr_v, sem1)
        cp2 = pltpu.async_copy(y_hbm.at[coli_v.at[j]], yc_v, sem2)
        cp1.wait()
        cp2.wait()

        def edge_body(i, a):
            out = []
            for jj in range(NV):
                sl = pl.ds(jj * 16, 16)
                hh = yr_v[i, sl] - yc_v[i, sl]
                hb_v[i, sl] = hh
                out.append(a[jj] + hh)
                out.append(a[NV + jj] + hh * hh)
            return tuple(out[::2]) + tuple(out[1::2])

        acc = lax.fori_loop(0, CH, edge_body, acc)
        off = pl.multiple_of(ebase + j * CH, 8)
        pltpu.sync_copy(hb_v, h_hbm.at[pl.ds(off, CH)])
        return acc

    acc = lax.fori_loop(0, NCH, chunk_body, init)
    for jj in range(NV):
        st_v[pl.ds(jj * 16, 16)] = acc[jj]
        st_v[pl.ds(D + jj * 16, 16)] = acc[NV + jj]
    pltpu.sync_copy(st_v, stats_hbm.at[wid])


@functools.partial(
    pl.kernel,
    out_type=(jax.ShapeDtypeStruct((NC, N, D), jnp.float32),
              jax.ShapeDtypeStruct((NC, N), jnp.float32)),
    mesh=_mesh,
    scratch_types=[
        pltpu.VMEM((NCH, CH), jnp.int32),
        pltpu.VMEM((CH, D), jnp.float32),
        pltpu.VMEM((2 * D,), jnp.float32),
        pltpu.VMEM((CH,), jnp.float32),
        pltpu.VMEM_SHARED((N, D), jnp.float32),
        pltpu.VMEM_SHARED((N,), jnp.float32),
        pltpu.SemaphoreType.DMA,
    ],
)
def _sc_pass2(h_hbm, row_hbm, st_hbm, zrow_hbm, zcnt_hbm, agg_hbm, cnt_hbm,
              rowi_v, hb_v, st_v, ones_v, acc_sh, cnt_sh, sem):
    c = lax.axis_index("c")
    s_ = lax.axis_index("s")
    wid = s_ * NC + c
    ebase = wid * EPW

    soff = pl.multiple_of(s_ * SB, 8)
    pltpu.sync_copy(zrow_hbm.at[pl.ds(soff, SB)], acc_sh.at[pl.ds(soff, SB)])

    @pl.when(s_ == NS - 1)
    def _():
        pltpu.sync_copy(zrow_hbm.at[pl.ds(NS * SB, SREM)],
                        acc_sh.at[pl.ds(NS * SB, SREM)])

    @pl.when(s_ == 0)
    def _():
        pltpu.sync_copy(zcnt_hbm, cnt_sh)

    pltpu.sync_copy(row_hbm.at[wid], rowi_v)
    pltpu.sync_copy(st_hbm, st_v)

    def ones_body(i, _):
        ones_v[pl.ds(i * 16, 16)] = jnp.ones((16,), jnp.float32)
        return 0

    lax.fori_loop(0, CH // 16, ones_body, 0)

    svec = [st_v[pl.ds(jj * 16, 16)] for jj in range(NV)]
    tvec = [st_v[pl.ds(D + jj * 16, 16)] for jj in range(NV)]

    plsc.subcore_barrier()

    def chunk_body(j, _):
        off = pl.multiple_of(ebase + j * CH, 8)
        pltpu.sync_copy(h_hbm.at[pl.ds(off, CH)], hb_v)

        def edge_body(i, carry):
            for jj in range(NV):
                sl = pl.ds(jj * 16, 16)
                v = hb_v[i, sl] * svec[jj] + tvec[jj]
                hb_v[i, sl] = jnp.maximum(v, 0.0)
            return carry

        lax.fori_loop(0, CH, edge_body, 0)
        pltpu.sync_copy(hb_v, acc_sh.at[rowi_v.at[j]], add=True)
        pltpu.sync_copy(ones_v, cnt_sh.at[rowi_v.at[j]], add=True)
        return 0

    lax.fori_loop(0, NCH, chunk_body, 0)
    plsc.subcore_barrier()

    pltpu.sync_copy(acc_sh.at[pl.ds(soff, SB)],
                    agg_hbm.at[c].at[pl.ds(soff, SB)])

    @pl.when(s_ == NS - 1)
    def _():
        pltpu.sync_copy(acc_sh.at[pl.ds(NS * SB, SREM)],
                        agg_hbm.at[c].at[pl.ds(NS * SB, SREM)])

    @pl.when(s_ == 0)
    def _():
        pltpu.sync_copy(cnt_sh, cnt_hbm.at[c])


def _tc_post_body(x_ref, agg_ref, cnt_ref, w2e_ref, bb2e_ref,
                  w1a_ref, g1a_ref, b1a_ref, w2a_ref, bb2a_ref, out_ref):
    agg_a = agg_ref[0] + agg_ref[1]
    cnt = cnt_ref[0, :] + cnt_ref[1, :]
    agg = jnp.dot(agg_a, w2e_ref[...], preferred_element_type=jnp.float32)
    agg = agg + cnt[:, None] * bb2e_ref[...]
    z = (jnp.dot(x_ref[...], w1a_ref[0], preferred_element_type=jnp.float32)
         + jnp.dot(agg, w1a_ref[1], preferred_element_type=jnp.float32))
    mean = jnp.mean(z, axis=0)
    zc = z - mean
    var = jnp.mean(zc * zc, axis=0)
    zb = zc * lax.rsqrt(var + EPS) * g1a_ref[...] + b1a_ref[...]
    zb = jnp.maximum(zb, 0.0)
    out_ref[...] = (jnp.dot(zb, w2a_ref[...], preferred_element_type=jnp.float32)
                    + bb2a_ref[...])


def _tc_post(x, agg, cnt, W2e, bb2e, W1a, g1a, b1a, W2a, bb2a):
    return pl.pallas_call(
        _tc_post_body,
        out_shape=jax.ShapeDtypeStruct((N, D), jnp.float32),
    )(x, agg, cnt, W2e, bb2e, W1a.reshape(2, D, D), g1a, b1a, W2a, bb2a)


@jax.jit
def kernel(x, edge_index, W1e, g1e, b1e, W2e, bb2e, W1a, g1a, b1a, W2a, bb2a):
    row = edge_index[0].astype(jnp.int32)
    col = edge_index[1].astype(jnp.int32)
    row3 = row.reshape(NW, NCH, CH)
    col3 = col.reshape(NW, NCH, CH)

    y = _tc_pre(x, W1e)
    h, stats = _sc_pass1(y, row3, col3)

    sums = jnp.sum(stats[:, :D], axis=0)
    sqs = jnp.sum(stats[:, D:], axis=0)
    mean = sums / E
    var = sqs / E - mean * mean
    s = g1e * lax.rsqrt(var + EPS)
    t = b1e - mean * s
    st = jnp.concatenate([s, t])

    zrow = jnp.zeros((N, D), jnp.float32)
    zcnt = jnp.zeros((N,), jnp.float32)
    agg_p, cnt_p = _sc_pass2(h, row3, st, zrow, zcnt)

    return _tc_post(x, agg_p, cnt_p, W2e, bb2e, W1a, g1a, b1a, W2a, bb2a)
